# baseline (speedup 1.0000x reference)
import functools

import jax
import jax.numpy as jnp
from jax import lax
from jax.experimental import pallas as pl
from jax.experimental.pallas import tpu as pltpu
from jax.experimental.pallas import tpu_sc as plsc

_INFO = plsc.get_sparse_core_info()
_NC, _NS = _INFO.num_cores, _INFO.num_subcores
_NW = _NC * _NS

_B = 4 * 8192
_D = 64
_B_PER_W = _B // _NW
_CHUNK = 512
_NCHUNK = _B_PER_W // _CHUNK


def _gather_body(pid_hbm, cos_hbm, sin_hbm, cos_out, sin_out,
                 idx_v, cos_rows, sin_rows, sem_c, sem_s):
    wid = lax.axis_index("s") * _NC + lax.axis_index("c")
    for c in range(_NCHUNK):
        base = wid * _B_PER_W + c * _CHUNK
        pltpu.sync_copy(pid_hbm.at[pl.ds(base, _CHUNK)], idx_v)
        cp_c = pltpu.async_copy(cos_hbm.at[idx_v], cos_rows, sem_c)
        cp_s = pltpu.async_copy(sin_hbm.at[idx_v], sin_rows, sem_s)
        cp_c.wait()
        pltpu.sync_copy(cos_rows, cos_out.at[pl.ds(base, _CHUNK)])
        cp_s.wait()
        pltpu.sync_copy(sin_rows, sin_out.at[pl.ds(base, _CHUNK)])


@functools.partial(jax.jit, static_argnames=())
def _rope_gather(position_ids_flat, cos_cached, sin_cached):
    mesh = plsc.VectorSubcoreMesh(core_axis_name="c", subcore_axis_name="s")
    k = pl.kernel(
        _gather_body,
        out_type=[
            jax.ShapeDtypeStruct((_B, _D), jnp.float32),
            jax.ShapeDtypeStruct((_B, _D), jnp.float32),
        ],
        mesh=mesh,
        scratch_types=[
            pltpu.VMEM((_CHUNK,), jnp.int32),
            pltpu.VMEM((_CHUNK, _D), jnp.float32),
            pltpu.VMEM((_CHUNK, _D), jnp.float32),
            pltpu.SemaphoreType.DMA,
            pltpu.SemaphoreType.DMA,
        ],
        compiler_params=pltpu.CompilerParams(use_tc_tiling_on_sc=False),
    )
    return k(position_ids_flat, cos_cached, sin_cached)


def kernel(x, position_ids, cos_cached, sin_cached):
    b, s = position_ids.shape
    pid = position_ids.reshape(b * s)
    cos, sin = _rope_gather(pid, cos_cached, sin_cached)
    cos = cos.reshape(b, s, _D).astype(x.dtype)
    sin = sin.reshape(b, s, _D).astype(x.dtype)
    return (cos, sin)
